# baseline (device time: 73891 ns/iter reference)
import jax
import jax.numpy as jnp
from jax import lax
from jax.experimental import pallas as pl
from jax.experimental.pallas import tpu as pltpu

N_Z = 4
NSB = 8

_ORDER = ((3, 2, 1, 0), (3, 0, 2, 1), (3, 0, 1, 2), (0, 1, 2, 3))


def kernel(x, dy):
    k, m = x.shape
    k2, f = dy.shape
    assert k == k2
    mc = m // N_Z
    fq = f // 4
    fqs = fq // NSB

    def body(x_ref, dy_ref, out_ref,
             dy_bf, part, acc,
             r_send, r_recv, l_send, l_recv,
             ag_acc, ag_rx, ag_ry, ag_rd,
             r_send_sems, r_recv_sems, l_send_sems, l_recv_sems,
             ag_send_sems, ag_recv_sems):
        my_x = lax.axis_index("x")
        my_y = lax.axis_index("y")
        my_z = lax.axis_index("z")
        q = my_x * 2 + my_y
        x_nbr = (1 - my_x, my_y, my_z)
        y_nbr = (my_x, 1 - my_y, my_z)
        d_nbr = (1 - my_x, 1 - my_y, my_z)

        barrier_sem = pltpu.get_barrier_semaphore()
        for dev in (x_nbr, y_nbr, d_nbr):
            pl.semaphore_signal(barrier_sem, inc=1, device_id=dev,
                                device_id_type=pl.DeviceIdType.MESH)

        @pl.when(my_z < N_Z - 1)
        def _():
            pl.semaphore_signal(barrier_sem, inc=1,
                                device_id=(my_x, my_y, my_z + 1),
                                device_id_type=pl.DeviceIdType.MESH)

        @pl.when(my_z > 0)
        def _():
            pl.semaphore_signal(barrier_sem, inc=1,
                                device_id=(my_x, my_y, my_z - 1),
                                device_id_type=pl.DeviceIdType.MESH)

        pl.semaphore_wait(barrier_sem, 4)

        @pl.when((my_z > 0) & (my_z < N_Z - 1))
        def _():
            pl.semaphore_wait(barrier_sem, 1)

        dy_bf[...] = dy_ref[:, pl.ds(q * fq, fq)].astype(jnp.bfloat16)

        def compute(i):
            c = jnp.where(
                my_z == 0, _ORDER[0][i],
                jnp.where(my_z == 1, _ORDER[1][i],
                          jnp.where(my_z == 2, _ORDER[2][i], _ORDER[3][i])))
            xs = x_ref[:, pl.ds(c * mc, mc)].astype(jnp.bfloat16)
            part[c] = lax.dot_general(
                xs, dy_bf[...],
                (((0,), (0,)), ((), ())),
                preferred_element_type=jnp.float32,
            )

        def make_rdma(send_buf, recv_buf, send_sems, recv_sems, c, sb, dst_z):
            return pltpu.make_async_remote_copy(
                src_ref=send_buf.at[c, sb],
                dst_ref=recv_buf.at[c, sb],
                send_sem=send_sems.at[c, sb],
                recv_sem=recv_sems.at[c, sb],
                device_id=(my_x, my_y, dst_z),
                device_id_type=pl.DeviceIdType.MESH,
            )

        def right_block(c, sb):
            sl = pl.ds(sb * fqs, fqs)

            @pl.when((c > my_z) & (my_z > 0))
            def _():
                make_rdma(r_send, r_recv, r_send_sems, r_recv_sems,
                          c, sb, my_z).wait_recv()
                r_send[c, sb] = (r_recv[c, sb].astype(jnp.float32)
                                 + part[c, :, sl]).astype(jnp.bfloat16)

            @pl.when((c > my_z) & (my_z == 0))
            def _():
                r_send[c, sb] = part[c, :, sl].astype(jnp.bfloat16)

            @pl.when(c > my_z)
            def _():
                make_rdma(r_send, r_recv, r_send_sems, r_recv_sems,
                          c, sb, my_z + 1).start()

        def left_block(c, sb):
            sl = pl.ds(sb * fqs, fqs)

            @pl.when((c < my_z) & (my_z < N_Z - 1))
            def _():
                make_rdma(l_send, l_recv, l_send_sems, l_recv_sems,
                          c, sb, my_z).wait_recv()
                l_send[c, sb] = (l_recv[c, sb].astype(jnp.float32)
                                 + part[c, :, sl]).astype(jnp.bfloat16)

            @pl.when((c < my_z) & (my_z == N_Z - 1))
            def _():
                l_send[c, sb] = part[c, :, sl].astype(jnp.bfloat16)

            @pl.when(c < my_z)
            def _():
                make_rdma(l_send, l_recv, l_send_sems, l_recv_sems,
                          c, sb, my_z - 1).start()

        def ag_rdma(dst_buf, sem_row, sb, dev):
            return pltpu.make_async_remote_copy(
                src_ref=ag_acc.at[sb],
                dst_ref=dst_buf.at[sb],
                send_sem=ag_send_sems.at[sem_row, sb],
                recv_sem=ag_recv_sems.at[sem_row, sb],
                device_id=dev,
                device_id_type=pl.DeviceIdType.MESH,
            )

        def keep_and_ag(sb):
            sl = pl.ds(sb * fqs, fqs)
            acc[:, sl] = part[my_z, :, sl]

            @pl.when(my_z > 0)
            def _():
                make_rdma(r_send, r_recv, r_send_sems, r_recv_sems,
                          my_z, sb, my_z).wait_recv()
                acc[:, sl] += r_recv[my_z, sb].astype(jnp.float32)

            @pl.when(my_z < N_Z - 1)
            def _():
                make_rdma(l_send, l_recv, l_send_sems, l_recv_sems,
                          my_z, sb, my_z).wait_recv()
                acc[:, sl] += l_recv[my_z, sb].astype(jnp.float32)

            ag_acc[sb] = acc[:, sl].astype(jnp.bfloat16)
            ag_rdma(ag_rx, 0, sb, x_nbr).start()
            ag_rdma(ag_ry, 1, sb, y_nbr).start()
            ag_rdma(ag_rd, 2, sb, d_nbr).start()
            out_ref[:, pl.ds(q * fq + sb * fqs, fqs)] = acc[:, sl]

        compute(0)
        right_block(3, 0)
        compute(1)
        left_block(0, 0)
        compute(2)
        right_block(2, 0)
        left_block(1, 0)
        compute(3)
        right_block(1, 0)
        left_block(2, 0)
        for sb in range(1, NSB):
            right_block(3, sb)
            left_block(0, sb)
            right_block(2, sb)
            left_block(1, sb)
            right_block(1, sb)
            left_block(2, sb)
        for sb in range(NSB):
            keep_and_ag(sb)

        qx = (1 - my_x) * 2 + my_y
        qy = my_x * 2 + (1 - my_y)
        qd = (1 - my_x) * 2 + (1 - my_y)
        for sb in range(NSB):
            ag_rdma(ag_rx, 0, sb, x_nbr).wait_recv()
            out_ref[:, pl.ds(qx * fq + sb * fqs, fqs)] = (
                ag_rx[sb].astype(jnp.float32))
            ag_rdma(ag_ry, 1, sb, y_nbr).wait_recv()
            out_ref[:, pl.ds(qy * fq + sb * fqs, fqs)] = (
                ag_ry[sb].astype(jnp.float32))
            ag_rdma(ag_rd, 2, sb, d_nbr).wait_recv()
            out_ref[:, pl.ds(qd * fq + sb * fqs, fqs)] = (
                ag_rd[sb].astype(jnp.float32))

        for c in range(N_Z):
            for sb in range(NSB):
                @pl.when(c > my_z)
                def _(c=c, sb=sb):
                    make_rdma(r_send, r_recv, r_send_sems, r_recv_sems,
                              c, sb, my_z).wait_send()

                @pl.when(c < my_z)
                def _(c=c, sb=sb):
                    make_rdma(l_send, l_recv, l_send_sems, l_recv_sems,
                              c, sb, my_z).wait_send()
        for sb in range(NSB):
            ag_rdma(ag_rx, 0, sb, x_nbr).wait_send()
            ag_rdma(ag_ry, 1, sb, y_nbr).wait_send()
            ag_rdma(ag_rd, 2, sb, d_nbr).wait_send()

    return pl.pallas_call(
        body,
        out_shape=jax.ShapeDtypeStruct((mc, f), jnp.float32),
        in_specs=[
            pl.BlockSpec(memory_space=pltpu.VMEM),
            pl.BlockSpec(memory_space=pltpu.VMEM),
        ],
        out_specs=pl.BlockSpec(memory_space=pltpu.VMEM),
        scratch_shapes=[
            pltpu.VMEM((k, fq), jnp.bfloat16),
            pltpu.VMEM((N_Z, mc, fq), jnp.float32),
            pltpu.VMEM((mc, fq), jnp.float32),
            pltpu.VMEM((N_Z, NSB, mc, fqs), jnp.bfloat16),
            pltpu.VMEM((N_Z, NSB, mc, fqs), jnp.bfloat16),
            pltpu.VMEM((N_Z, NSB, mc, fqs), jnp.bfloat16),
            pltpu.VMEM((N_Z, NSB, mc, fqs), jnp.bfloat16),
            pltpu.VMEM((NSB, mc, fqs), jnp.bfloat16),
            pltpu.VMEM((NSB, mc, fqs), jnp.bfloat16),
            pltpu.VMEM((NSB, mc, fqs), jnp.bfloat16),
            pltpu.VMEM((NSB, mc, fqs), jnp.bfloat16),
            pltpu.SemaphoreType.DMA((N_Z, NSB)),
            pltpu.SemaphoreType.DMA((N_Z, NSB)),
            pltpu.SemaphoreType.DMA((N_Z, NSB)),
            pltpu.SemaphoreType.DMA((N_Z, NSB)),
            pltpu.SemaphoreType.DMA((3, NSB)),
            pltpu.SemaphoreType.DMA((3, NSB)),
        ],
        compiler_params=pltpu.CompilerParams(
            collective_id=0,
            vmem_limit_bytes=100 * 1024 * 1024,
        ),
    )(x, dy)


# device time: 59784 ns/iter; 1.2360x vs baseline; 1.2360x over previous
import jax
import jax.numpy as jnp
from jax import lax
from jax.experimental import pallas as pl
from jax.experimental.pallas import tpu as pltpu

N_Z = 4
NSB = 4

_ORDER = ((3, 2, 1, 0), (3, 0, 2, 1), (3, 0, 1, 2), (0, 1, 2, 3))


def kernel(x, dy):
    k, m = x.shape
    k2, f = dy.shape
    assert k == k2
    mc = m // N_Z
    fq = f // 4
    fqs = fq // NSB

    def body(x_ref, dy_ref, out_ref,
             dy_bf, part, acc,
             r_send, r_recv, l_send, l_recv,
             ag_acc, ag_rx, ag_ry, ag_rd,
             r_send_sems, r_recv_sems, l_send_sems, l_recv_sems,
             ag_send_sems, ag_recv_sems):
        my_x = lax.axis_index("x")
        my_y = lax.axis_index("y")
        my_z = lax.axis_index("z")
        q = my_x * 2 + my_y
        x_nbr = (1 - my_x, my_y, my_z)
        y_nbr = (my_x, 1 - my_y, my_z)
        d_nbr = (1 - my_x, 1 - my_y, my_z)

        barrier_sem = pltpu.get_barrier_semaphore()
        for dev in (x_nbr, y_nbr, d_nbr):
            pl.semaphore_signal(barrier_sem, inc=1, device_id=dev,
                                device_id_type=pl.DeviceIdType.MESH)

        @pl.when(my_z < N_Z - 1)
        def _():
            pl.semaphore_signal(barrier_sem, inc=1,
                                device_id=(my_x, my_y, my_z + 1),
                                device_id_type=pl.DeviceIdType.MESH)

        @pl.when(my_z > 0)
        def _():
            pl.semaphore_signal(barrier_sem, inc=1,
                                device_id=(my_x, my_y, my_z - 1),
                                device_id_type=pl.DeviceIdType.MESH)

        pl.semaphore_wait(barrier_sem, 4)

        @pl.when((my_z > 0) & (my_z < N_Z - 1))
        def _():
            pl.semaphore_wait(barrier_sem, 1)

        dy_bf[...] = dy_ref[:, pl.ds(q * fq, fq)].astype(jnp.bfloat16)

        def compute(i):
            c = jnp.where(
                my_z == 0, _ORDER[0][i],
                jnp.where(my_z == 1, _ORDER[1][i],
                          jnp.where(my_z == 2, _ORDER[2][i], _ORDER[3][i])))
            xs = x_ref[:, pl.ds(c * mc, mc)].astype(jnp.bfloat16)
            part[c] = lax.dot_general(
                xs, dy_bf[...],
                (((0,), (0,)), ((), ())),
                preferred_element_type=jnp.float32,
            )

        def make_rdma(send_buf, recv_buf, send_sems, recv_sems, c, sb, dst_z):
            return pltpu.make_async_remote_copy(
                src_ref=send_buf.at[c, sb],
                dst_ref=recv_buf.at[c, sb],
                send_sem=send_sems.at[c, sb],
                recv_sem=recv_sems.at[c, sb],
                device_id=(my_x, my_y, dst_z),
                device_id_type=pl.DeviceIdType.MESH,
            )

        def right_block(c, sb):
            sl = pl.ds(sb * fqs, fqs)

            @pl.when((c > my_z) & (my_z > 0))
            def _():
                make_rdma(r_send, r_recv, r_send_sems, r_recv_sems,
                          c, sb, my_z).wait_recv()
                r_send[c, sb] = (r_recv[c, sb].astype(jnp.float32)
                                 + part[c, :, sl]).astype(jnp.bfloat16)

            @pl.when((c > my_z) & (my_z == 0))
            def _():
                r_send[c, sb] = part[c, :, sl].astype(jnp.bfloat16)

            @pl.when(c > my_z)
            def _():
                make_rdma(r_send, r_recv, r_send_sems, r_recv_sems,
                          c, sb, my_z + 1).start()

        def left_block(c, sb):
            sl = pl.ds(sb * fqs, fqs)

            @pl.when((c < my_z) & (my_z < N_Z - 1))
            def _():
                make_rdma(l_send, l_recv, l_send_sems, l_recv_sems,
                          c, sb, my_z).wait_recv()
                l_send[c, sb] = (l_recv[c, sb].astype(jnp.float32)
                                 + part[c, :, sl]).astype(jnp.bfloat16)

            @pl.when((c < my_z) & (my_z == N_Z - 1))
            def _():
                l_send[c, sb] = part[c, :, sl].astype(jnp.bfloat16)

            @pl.when(c < my_z)
            def _():
                make_rdma(l_send, l_recv, l_send_sems, l_recv_sems,
                          c, sb, my_z - 1).start()

        def ag_rdma(dst_buf, sem_row, sb, dev):
            return pltpu.make_async_remote_copy(
                src_ref=ag_acc.at[sb],
                dst_ref=dst_buf.at[sb],
                send_sem=ag_send_sems.at[sem_row, sb],
                recv_sem=ag_recv_sems.at[sem_row, sb],
                device_id=dev,
                device_id_type=pl.DeviceIdType.MESH,
            )

        def keep_and_ag(sb):
            sl = pl.ds(sb * fqs, fqs)
            acc[:, sl] = part[my_z, :, sl]

            @pl.when(my_z > 0)
            def _():
                make_rdma(r_send, r_recv, r_send_sems, r_recv_sems,
                          my_z, sb, my_z).wait_recv()
                acc[:, sl] += r_recv[my_z, sb].astype(jnp.float32)

            @pl.when(my_z < N_Z - 1)
            def _():
                make_rdma(l_send, l_recv, l_send_sems, l_recv_sems,
                          my_z, sb, my_z).wait_recv()
                acc[:, sl] += l_recv[my_z, sb].astype(jnp.float32)

            ag_acc[sb] = acc[:, sl].astype(jnp.bfloat16)
            ag_rdma(ag_rx, 0, sb, x_nbr).start()
            ag_rdma(ag_ry, 1, sb, y_nbr).start()
            ag_rdma(ag_rd, 2, sb, d_nbr).start()
            out_ref[:, pl.ds(q * fq + sb * fqs, fqs)] = acc[:, sl]

        compute(0)
        right_block(3, 0)
        compute(1)
        left_block(0, 0)
        right_block(3, 1)
        left_block(0, 1)
        compute(2)
        right_block(2, 0)
        left_block(1, 0)
        right_block(1, 0)
        left_block(2, 0)
        compute(3)
        right_block(3, 2)
        left_block(0, 2)
        right_block(2, 1)
        left_block(1, 1)
        right_block(1, 1)
        left_block(2, 1)
        right_block(3, 3)
        left_block(0, 3)
        right_block(2, 2)
        left_block(1, 2)
        right_block(1, 2)
        left_block(2, 2)
        right_block(2, 3)
        left_block(1, 3)
        right_block(1, 3)
        left_block(2, 3)
        for sb in range(NSB):
            keep_and_ag(sb)

        qx = (1 - my_x) * 2 + my_y
        qy = my_x * 2 + (1 - my_y)
        qd = (1 - my_x) * 2 + (1 - my_y)
        for sb in range(NSB):
            ag_rdma(ag_rx, 0, sb, x_nbr).wait_recv()
            out_ref[:, pl.ds(qx * fq + sb * fqs, fqs)] = (
                ag_rx[sb].astype(jnp.float32))
            ag_rdma(ag_ry, 1, sb, y_nbr).wait_recv()
            out_ref[:, pl.ds(qy * fq + sb * fqs, fqs)] = (
                ag_ry[sb].astype(jnp.float32))
            ag_rdma(ag_rd, 2, sb, d_nbr).wait_recv()
            out_ref[:, pl.ds(qd * fq + sb * fqs, fqs)] = (
                ag_rd[sb].astype(jnp.float32))

        for c in range(N_Z):
            for sb in range(NSB):
                @pl.when(c > my_z)
                def _(c=c, sb=sb):
                    make_rdma(r_send, r_recv, r_send_sems, r_recv_sems,
                              c, sb, my_z).wait_send()

                @pl.when(c < my_z)
                def _(c=c, sb=sb):
                    make_rdma(l_send, l_recv, l_send_sems, l_recv_sems,
                              c, sb, my_z).wait_send()
        for sb in range(NSB):
            ag_rdma(ag_rx, 0, sb, x_nbr).wait_send()
            ag_rdma(ag_ry, 1, sb, y_nbr).wait_send()
            ag_rdma(ag_rd, 2, sb, d_nbr).wait_send()

    return pl.pallas_call(
        body,
        out_shape=jax.ShapeDtypeStruct((mc, f), jnp.float32),
        in_specs=[
            pl.BlockSpec(memory_space=pltpu.VMEM),
            pl.BlockSpec(memory_space=pltpu.VMEM),
        ],
        out_specs=pl.BlockSpec(memory_space=pltpu.VMEM),
        scratch_shapes=[
            pltpu.VMEM((k, fq), jnp.bfloat16),
            pltpu.VMEM((N_Z, mc, fq), jnp.float32),
            pltpu.VMEM((mc, fq), jnp.float32),
            pltpu.VMEM((N_Z, NSB, mc, fqs), jnp.bfloat16),
            pltpu.VMEM((N_Z, NSB, mc, fqs), jnp.bfloat16),
            pltpu.VMEM((N_Z, NSB, mc, fqs), jnp.bfloat16),
            pltpu.VMEM((N_Z, NSB, mc, fqs), jnp.bfloat16),
            pltpu.VMEM((NSB, mc, fqs), jnp.bfloat16),
            pltpu.VMEM((NSB, mc, fqs), jnp.bfloat16),
            pltpu.VMEM((NSB, mc, fqs), jnp.bfloat16),
            pltpu.VMEM((NSB, mc, fqs), jnp.bfloat16),
            pltpu.SemaphoreType.DMA((N_Z, NSB)),
            pltpu.SemaphoreType.DMA((N_Z, NSB)),
            pltpu.SemaphoreType.DMA((N_Z, NSB)),
            pltpu.SemaphoreType.DMA((N_Z, NSB)),
            pltpu.SemaphoreType.DMA((3, NSB)),
            pltpu.SemaphoreType.DMA((3, NSB)),
        ],
        compiler_params=pltpu.CompilerParams(
            collective_id=0,
            vmem_limit_bytes=100 * 1024 * 1024,
        ),
    )(x, dy)


# device time: 54533 ns/iter; 1.3550x vs baseline; 1.0963x over previous
import jax
import jax.numpy as jnp
from jax import lax
from jax.experimental import pallas as pl
from jax.experimental.pallas import tpu as pltpu

N_Z = 4
NSB = 2

_ORDER = ((3, 2, 1, 0), (3, 0, 2, 1), (3, 0, 1, 2), (0, 1, 2, 3))


def kernel(x, dy):
    k, m = x.shape
    k2, f = dy.shape
    assert k == k2
    mc = m // N_Z
    fq = f // 4
    fqs = fq // NSB

    def body(x_ref, dy_ref, out_ref,
             dy_bf, part, acc,
             r_send, r_recv, l_send, l_recv,
             ag_acc, ag_rx, ag_ry, ag_rd,
             r_send_sems, r_recv_sems, l_send_sems, l_recv_sems,
             ag_send_sems, ag_recv_sems):
        my_x = lax.axis_index("x")
        my_y = lax.axis_index("y")
        my_z = lax.axis_index("z")
        q = my_x * 2 + my_y
        x_nbr = (1 - my_x, my_y, my_z)
        y_nbr = (my_x, 1 - my_y, my_z)
        d_nbr = (1 - my_x, 1 - my_y, my_z)

        barrier_sem = pltpu.get_barrier_semaphore()
        for dev in (x_nbr, y_nbr, d_nbr):
            pl.semaphore_signal(barrier_sem, inc=1, device_id=dev,
                                device_id_type=pl.DeviceIdType.MESH)

        @pl.when(my_z < N_Z - 1)
        def _():
            pl.semaphore_signal(barrier_sem, inc=1,
                                device_id=(my_x, my_y, my_z + 1),
                                device_id_type=pl.DeviceIdType.MESH)

        @pl.when(my_z > 0)
        def _():
            pl.semaphore_signal(barrier_sem, inc=1,
                                device_id=(my_x, my_y, my_z - 1),
                                device_id_type=pl.DeviceIdType.MESH)

        pl.semaphore_wait(barrier_sem, 4)

        @pl.when((my_z > 0) & (my_z < N_Z - 1))
        def _():
            pl.semaphore_wait(barrier_sem, 1)

        dy_bf[...] = dy_ref[:, pl.ds(q * fq, fq)].astype(jnp.bfloat16)

        def compute(i):
            c = jnp.where(
                my_z == 0, _ORDER[0][i],
                jnp.where(my_z == 1, _ORDER[1][i],
                          jnp.where(my_z == 2, _ORDER[2][i], _ORDER[3][i])))
            xs = x_ref[:, pl.ds(c * mc, mc)].astype(jnp.bfloat16)
            part[c] = lax.dot_general(
                xs, dy_bf[...],
                (((0,), (0,)), ((), ())),
                preferred_element_type=jnp.float32,
            )

        def make_rdma(send_buf, recv_buf, send_sems, recv_sems, c, sb, dst_z):
            return pltpu.make_async_remote_copy(
                src_ref=send_buf.at[c, sb],
                dst_ref=recv_buf.at[c, sb],
                send_sem=send_sems.at[c, sb],
                recv_sem=recv_sems.at[c, sb],
                device_id=(my_x, my_y, dst_z),
                device_id_type=pl.DeviceIdType.MESH,
            )

        def right_block(c, sb):
            sl = pl.ds(sb * fqs, fqs)

            @pl.when((c > my_z) & (my_z > 0))
            def _():
                make_rdma(r_send, r_recv, r_send_sems, r_recv_sems,
                          c, sb, my_z).wait_recv()
                r_send[c, sb] = (r_recv[c, sb].astype(jnp.float32)
                                 + part[c, :, sl]).astype(jnp.bfloat16)

            @pl.when((c > my_z) & (my_z == 0))
            def _():
                r_send[c, sb] = part[c, :, sl].astype(jnp.bfloat16)

            @pl.when(c > my_z)
            def _():
                make_rdma(r_send, r_recv, r_send_sems, r_recv_sems,
                          c, sb, my_z + 1).start()

        def left_block(c, sb):
            sl = pl.ds(sb * fqs, fqs)

            @pl.when((c < my_z) & (my_z < N_Z - 1))
            def _():
                make_rdma(l_send, l_recv, l_send_sems, l_recv_sems,
                          c, sb, my_z).wait_recv()
                l_send[c, sb] = (l_recv[c, sb].astype(jnp.float32)
                                 + part[c, :, sl]).astype(jnp.bfloat16)

            @pl.when((c < my_z) & (my_z == N_Z - 1))
            def _():
                l_send[c, sb] = part[c, :, sl].astype(jnp.bfloat16)

            @pl.when(c < my_z)
            def _():
                make_rdma(l_send, l_recv, l_send_sems, l_recv_sems,
                          c, sb, my_z - 1).start()

        def ag_rdma(dst_buf, sem_row, sb, dev):
            return pltpu.make_async_remote_copy(
                src_ref=ag_acc.at[sb],
                dst_ref=dst_buf.at[sb],
                send_sem=ag_send_sems.at[sem_row, sb],
                recv_sem=ag_recv_sems.at[sem_row, sb],
                device_id=dev,
                device_id_type=pl.DeviceIdType.MESH,
            )

        def keep_and_ag(sb):
            sl = pl.ds(sb * fqs, fqs)
            acc[:, sl] = part[my_z, :, sl]

            @pl.when(my_z > 0)
            def _():
                make_rdma(r_send, r_recv, r_send_sems, r_recv_sems,
                          my_z, sb, my_z).wait_recv()
                acc[:, sl] += r_recv[my_z, sb].astype(jnp.float32)

            @pl.when(my_z < N_Z - 1)
            def _():
                make_rdma(l_send, l_recv, l_send_sems, l_recv_sems,
                          my_z, sb, my_z).wait_recv()
                acc[:, sl] += l_recv[my_z, sb].astype(jnp.float32)

            ag_acc[sb] = acc[:, sl].astype(jnp.bfloat16)
            ag_rdma(ag_rx, 0, sb, x_nbr).start()
            ag_rdma(ag_ry, 1, sb, y_nbr).start()
            ag_rdma(ag_rd, 2, sb, d_nbr).start()
            out_ref[:, pl.ds(q * fq + sb * fqs, fqs)] = acc[:, sl]

        compute(0)
        right_block(3, 0)
        compute(1)
        left_block(0, 0)
        right_block(3, 1)
        left_block(0, 1)
        compute(2)
        right_block(2, 0)
        left_block(1, 0)
        right_block(1, 0)
        left_block(2, 0)
        compute(3)
        right_block(2, 1)
        left_block(1, 1)
        right_block(1, 1)
        left_block(2, 1)
        for sb in range(NSB):
            keep_and_ag(sb)

        qx = (1 - my_x) * 2 + my_y
        qy = my_x * 2 + (1 - my_y)
        qd = (1 - my_x) * 2 + (1 - my_y)
        for sb in range(NSB):
            ag_rdma(ag_rx, 0, sb, x_nbr).wait_recv()
            out_ref[:, pl.ds(qx * fq + sb * fqs, fqs)] = (
                ag_rx[sb].astype(jnp.float32))
            ag_rdma(ag_ry, 1, sb, y_nbr).wait_recv()
            out_ref[:, pl.ds(qy * fq + sb * fqs, fqs)] = (
                ag_ry[sb].astype(jnp.float32))
            ag_rdma(ag_rd, 2, sb, d_nbr).wait_recv()
            out_ref[:, pl.ds(qd * fq + sb * fqs, fqs)] = (
                ag_rd[sb].astype(jnp.float32))

        for c in range(N_Z):
            for sb in range(NSB):
                @pl.when(c > my_z)
                def _(c=c, sb=sb):
                    make_rdma(r_send, r_recv, r_send_sems, r_recv_sems,
                              c, sb, my_z).wait_send()

                @pl.when(c < my_z)
                def _(c=c, sb=sb):
                    make_rdma(l_send, l_recv, l_send_sems, l_recv_sems,
                              c, sb, my_z).wait_send()
        for sb in range(NSB):
            ag_rdma(ag_rx, 0, sb, x_nbr).wait_send()
            ag_rdma(ag_ry, 1, sb, y_nbr).wait_send()
            ag_rdma(ag_rd, 2, sb, d_nbr).wait_send()

    return pl.pallas_call(
        body,
        out_shape=jax.ShapeDtypeStruct((mc, f), jnp.float32),
        in_specs=[
            pl.BlockSpec(memory_space=pltpu.VMEM),
            pl.BlockSpec(memory_space=pltpu.VMEM),
        ],
        out_specs=pl.BlockSpec(memory_space=pltpu.VMEM),
        scratch_shapes=[
            pltpu.VMEM((k, fq), jnp.bfloat16),
            pltpu.VMEM((N_Z, mc, fq), jnp.float32),
            pltpu.VMEM((mc, fq), jnp.float32),
            pltpu.VMEM((N_Z, NSB, mc, fqs), jnp.bfloat16),
            pltpu.VMEM((N_Z, NSB, mc, fqs), jnp.bfloat16),
            pltpu.VMEM((N_Z, NSB, mc, fqs), jnp.bfloat16),
            pltpu.VMEM((N_Z, NSB, mc, fqs), jnp.bfloat16),
            pltpu.VMEM((NSB, mc, fqs), jnp.bfloat16),
            pltpu.VMEM((NSB, mc, fqs), jnp.bfloat16),
            pltpu.VMEM((NSB, mc, fqs), jnp.bfloat16),
            pltpu.VMEM((NSB, mc, fqs), jnp.bfloat16),
            pltpu.SemaphoreType.DMA((N_Z, NSB)),
            pltpu.SemaphoreType.DMA((N_Z, NSB)),
            pltpu.SemaphoreType.DMA((N_Z, NSB)),
            pltpu.SemaphoreType.DMA((N_Z, NSB)),
            pltpu.SemaphoreType.DMA((3, NSB)),
            pltpu.SemaphoreType.DMA((3, NSB)),
        ],
        compiler_params=pltpu.CompilerParams(
            collective_id=0,
            vmem_limit_bytes=100 * 1024 * 1024,
        ),
    )(x, dy)
